# Initial kernel scaffold; baseline (speedup 1.0000x reference)
import jax
import jax.numpy as jnp
from jax.experimental import pallas as pl


def kernel(x, embed_table, W_fc, b_fc):
    raise NotImplementedError("write your pallas kernel here")


# speedup vs baseline: 7.4710x; 7.4710x over previous
import functools

import jax
import jax.numpy as jnp
from jax import lax
from jax.experimental import pallas as pl
from jax.experimental.pallas import tpu as pltpu
from jax.experimental.pallas import tpu_sc as plsc

VOCAB = 100000
D = 64
NF = 2
B = 4096
L = 50
NCLS = 2
LANES = 16

NC, NS = 2, 16
NW = NC * NS
BPW = B // NW

_PROJ_BLK = 2000


def _proj_body(emb_ref, w0_ref, w1_ref, o0_ref, o1_ref):
    e = emb_ref[...]
    o0_ref[...] = jnp.dot(e, w0_ref[...], preferred_element_type=jnp.float32)
    o1_ref[...] = jnp.dot(e, w1_ref[...], preferred_element_type=jnp.float32)


def _project(embed_table, w0p, w1p):
    grid = VOCAB // _PROJ_BLK
    return pl.pallas_call(
        _proj_body,
        grid=(grid,),
        in_specs=[
            pl.BlockSpec((_PROJ_BLK, D), lambda i: (i, 0)),
            pl.BlockSpec((D, LANES), lambda i: (0, 0)),
            pl.BlockSpec((D, LANES), lambda i: (0, 0)),
        ],
        out_specs=[
            pl.BlockSpec((_PROJ_BLK, LANES), lambda i: (i, 0)),
            pl.BlockSpec((_PROJ_BLK, LANES), lambda i: (i, 0)),
        ],
        out_shape=[
            jax.ShapeDtypeStruct((VOCAB, LANES), jnp.float32),
            jax.ShapeDtypeStruct((VOCAB, LANES), jnp.float32),
        ],
    )(embed_table, w0p, w1p)


def _pool_body(t0_hbm, t1_hbm, x_hbm, bias_hbm, out_hbm,
               idx_v, rows_v, out_v, bias_v, sem0, sem1):
    wid = lax.axis_index("s") * NC + lax.axis_index("c")
    base = wid * BPW
    pltpu.sync_copy(x_hbm.at[0, pl.ds(base, BPW)], idx_v.at[0])
    pltpu.sync_copy(x_hbm.at[1, pl.ds(base, BPW)], idx_v.at[1])
    pltpu.sync_copy(bias_hbm, bias_v)
    bias = bias_v[...]

    def start(b, s, sem):
        c0 = pltpu.async_copy(t0_hbm.at[idx_v.at[0, b]],
                              rows_v.at[s, pl.ds(0, L)], sem)
        c1 = pltpu.async_copy(t1_hbm.at[idx_v.at[1, b]],
                              rows_v.at[s, pl.ds(L, L)], sem)
        return c0, c1

    def wait(b, s, sem):
        c0, c1 = (pltpu.make_async_copy(t0_hbm.at[idx_v.at[0, b]],
                                        rows_v.at[s, pl.ds(0, L)], sem),
                  pltpu.make_async_copy(t1_hbm.at[idx_v.at[1, b]],
                                        rows_v.at[s, pl.ds(L, L)], sem))
        c0.wait()
        c1.wait()

    def reduce_into(b, s):
        def acc_body(j, acc):
            return acc + rows_v[s, j]
        acc = lax.fori_loop(0, NF * L, acc_body,
                            jnp.zeros((LANES,), jnp.float32))
        out_v[b] = acc * jnp.float32(1.0 / L) + bias

    start(0, 0, sem0)

    def body(b, carry):
        s = lax.rem(b, 2)
        sn = lax.rem(b + 1, 2)

        @pl.when(b + 1 < BPW)
        def _():
            @pl.when(sn == 0)
            def _():
                start(b + 1, 0, sem0)

            @pl.when(sn == 1)
            def _():
                start(b + 1, 1, sem1)

        @pl.when(s == 0)
        def _():
            wait(b, 0, sem0)
            reduce_into(b, 0)

        @pl.when(s == 1)
        def _():
            wait(b, 1, sem1)
            reduce_into(b, 1)

        return carry

    lax.fori_loop(0, BPW, body, 0)
    pltpu.sync_copy(out_v, out_hbm.at[pl.ds(base, BPW)])


_pool = functools.partial(
    pl.kernel,
    out_type=jax.ShapeDtypeStruct((B, LANES), jnp.float32),
    mesh=plsc.VectorSubcoreMesh(core_axis_name="c", subcore_axis_name="s"),
    scratch_types=[
        pltpu.VMEM((NF, BPW, L), jnp.int32),
        pltpu.VMEM((2, NF * L, LANES), jnp.float32),
        pltpu.VMEM((BPW, LANES), jnp.float32),
        pltpu.VMEM((LANES,), jnp.float32),
        pltpu.SemaphoreType.DMA,
        pltpu.SemaphoreType.DMA,
    ],
    compiler_params=pltpu.CompilerParams(use_tc_tiling_on_sc=False),
)(_pool_body)


def kernel(x, embed_table, W_fc, b_fc):
    w0p = jnp.zeros((D, LANES), jnp.float32).at[:, :NCLS].set(W_fc[:D])
    w1p = jnp.zeros((D, LANES), jnp.float32).at[:, :NCLS].set(W_fc[D:])
    bpad = jnp.zeros((LANES,), jnp.float32).at[:NCLS].set(b_fc)
    t0, t1 = _project(embed_table, w0p, w1p)
    out16 = _pool(t0, t1, x, bpad)
    return out16[:, :NCLS]


# speedup vs baseline: 14.4766x; 1.9377x over previous
import functools

import jax
import jax.numpy as jnp
from jax import lax
from jax.experimental import pallas as pl
from jax.experimental.pallas import tpu as pltpu
from jax.experimental.pallas import tpu_sc as plsc

VOCAB = 100000
D = 64
NF = 2
B = 4096
L = 50
NCLS = 2
LANES = 16
PACK = 128 // LANES

NC, NS = 2, 16
NW = NC * NS
BPW = B // NW
CB = 2
NCHUNK = BPW // CB
ROWS = CB * NF * L

_PROJ_BLK = 1250
_VR = VOCAB // PACK


def _proj_body(emb_ref, w0_ref, w1_ref, o0_ref, o1_ref):
    e = emb_ref[...]
    o0_ref[...] = jnp.dot(e, w0_ref[...], preferred_element_type=jnp.float32)
    o1_ref[...] = jnp.dot(e, w1_ref[...], preferred_element_type=jnp.float32)


def _project(e512, wbd0, wbd1):
    return pl.pallas_call(
        _proj_body,
        out_shape=[
            jax.ShapeDtypeStruct((_VR, 128), jnp.float32),
            jax.ShapeDtypeStruct((_VR, 128), jnp.float32),
        ],
    )(e512, wbd0, wbd1)


def _pool_body(t0_hbm, t1_hbm, x_hbm, bias_hbm, out_hbm,
               idx_v, rows_v, out_v, bias_v, sem0, sem1):
    wid = lax.axis_index("s") * NC + lax.axis_index("c")
    base = wid * BPW
    pltpu.sync_copy(x_hbm.at[0, pl.ds(wid * NCHUNK, NCHUNK)], idx_v.at[0])
    pltpu.sync_copy(x_hbm.at[1, pl.ds(wid * NCHUNK, NCHUNK)], idx_v.at[1])
    pltpu.sync_copy(bias_hbm, bias_v)
    bias = bias_v[...]
    scale = jnp.float32(1.0 / L)

    def start(c, s, sem):
        pltpu.async_copy(t0_hbm.at[idx_v.at[0, c]], rows_v.at[s, 0], sem)
        pltpu.async_copy(t1_hbm.at[idx_v.at[1, c]], rows_v.at[s, 1], sem)

    def wait(c, s, sem):
        pltpu.make_async_copy(t0_hbm.at[idx_v.at[0, c]],
                              rows_v.at[s, 0], sem).wait()
        pltpu.make_async_copy(t1_hbm.at[idx_v.at[1, c]],
                              rows_v.at[s, 1], sem).wait()

    def reduce_chunk(c, s):
        for u in range(CB):
            accs = [jnp.zeros((LANES,), jnp.float32) for _ in range(4)]
            for half in range(NF):
                for j in range(L):
                    accs[j % 4] = accs[j % 4] + rows_v[s, half, u * L + j]
            acc = (accs[0] + accs[1]) + (accs[2] + accs[3])
            out_v[c * CB + u] = acc * scale + bias

    start(0, 0, sem0)

    def body(c, carry):
        sn = lax.rem(c + 1, 2)

        @pl.when(c + 1 < NCHUNK)
        def _():
            @pl.when(sn == 0)
            def _():
                start(c + 1, 0, sem0)

            @pl.when(sn == 1)
            def _():
                start(c + 1, 1, sem1)

        @pl.when(lax.rem(c, 2) == 0)
        def _():
            wait(c, 0, sem0)
            reduce_chunk(c, 0)

        @pl.when(lax.rem(c, 2) == 1)
        def _():
            wait(c, 1, sem1)
            reduce_chunk(c, 1)

        return carry

    lax.fori_loop(0, NCHUNK, body, 0)
    pltpu.sync_copy(out_v, out_hbm.at[pl.ds(base, BPW)])


_pool = functools.partial(
    pl.kernel,
    out_type=jax.ShapeDtypeStruct((B, LANES), jnp.float32),
    mesh=plsc.VectorSubcoreMesh(core_axis_name="c", subcore_axis_name="s"),
    scratch_types=[
        pltpu.VMEM((NF, NCHUNK, CB * L), jnp.int32),
        pltpu.VMEM((2, NF, CB * L, LANES), jnp.float32),
        pltpu.VMEM((BPW, LANES), jnp.float32),
        pltpu.VMEM((LANES,), jnp.float32),
        pltpu.SemaphoreType.DMA,
        pltpu.SemaphoreType.DMA,
    ],
    compiler_params=pltpu.CompilerParams(use_tc_tiling_on_sc=False),
)(_pool_body)


def kernel(x, embed_table, W_fc, b_fc):
    w0p = jnp.zeros((D, LANES), jnp.float32).at[:, :NCLS].set(W_fc[:D])
    w1p = jnp.zeros((D, LANES), jnp.float32).at[:, :NCLS].set(W_fc[D:])
    eye8 = jnp.eye(PACK, dtype=jnp.float32)
    wbd0 = jnp.kron(eye8, w0p)
    wbd1 = jnp.kron(eye8, w1p)
    bpad = jnp.zeros((LANES,), jnp.float32).at[:NCLS].set(b_fc)
    e512 = embed_table.reshape(_VR, PACK * D)
    t0p, t1p = _project(e512, wbd0, wbd1)
    t0 = t0p.reshape(VOCAB, LANES)
    t1 = t1p.reshape(VOCAB, LANES)
    x3 = x.reshape(NF, NW * NCHUNK, CB * L)
    out16 = _pool(t0, t1, x3, bpad)
    return out16[:, :NCLS]


# speedup vs baseline: 15.9975x; 1.1051x over previous
import functools

import jax
import jax.numpy as jnp
from jax import lax
from jax.experimental import pallas as pl
from jax.experimental.pallas import tpu as pltpu
from jax.experimental.pallas import tpu_sc as plsc

VOCAB = 100000
D = 64
NF = 2
B = 4096
L = 50
NCLS = 2
LANES = 16
PACK = 128 // LANES

NC, NS = 2, 16
NW = NC * NS
BPW = B // NW
CB = 2
NCHUNK = BPW // CB
ROWS = CB * NF * L
NBUF = 4

_PROJ_BLK = 1250
_VR = VOCAB // PACK


def _proj_body(emb_ref, w0_ref, w1_ref, o0_ref, o1_ref):
    e = emb_ref[...]
    o0_ref[...] = jnp.dot(e, w0_ref[...], preferred_element_type=jnp.float32)
    o1_ref[...] = jnp.dot(e, w1_ref[...], preferred_element_type=jnp.float32)


def _project(e512, wbd0, wbd1):
    return pl.pallas_call(
        _proj_body,
        out_shape=[
            jax.ShapeDtypeStruct((_VR, 128), jnp.float32),
            jax.ShapeDtypeStruct((_VR, 128), jnp.float32),
        ],
    )(e512, wbd0, wbd1)


def _pool_body(t0_hbm, t1_hbm, x_hbm, bias_hbm, out_hbm,
               idx_v, rows_v, out_v, bias_v, sem0, sem1, sem2, sem3):
    wid = lax.axis_index("s") * NC + lax.axis_index("c")
    base = wid * BPW
    pltpu.sync_copy(x_hbm.at[0, pl.ds(wid * NCHUNK, NCHUNK)], idx_v.at[0])
    pltpu.sync_copy(x_hbm.at[1, pl.ds(wid * NCHUNK, NCHUNK)], idx_v.at[1])
    pltpu.sync_copy(bias_hbm, bias_v)
    bias = bias_v[...]
    scale = jnp.float32(1.0 / L)

    sems = (sem0, sem1, sem2, sem3)

    def start(c, s):
        pltpu.async_copy(t0_hbm.at[idx_v.at[0, c]], rows_v.at[s, 0], sems[s])
        pltpu.async_copy(t1_hbm.at[idx_v.at[1, c]], rows_v.at[s, 1], sems[s])

    def wait(c, s):
        pltpu.make_async_copy(t0_hbm.at[idx_v.at[0, c]],
                              rows_v.at[s, 0], sems[s]).wait()
        pltpu.make_async_copy(t1_hbm.at[idx_v.at[1, c]],
                              rows_v.at[s, 1], sems[s]).wait()

    def reduce_chunk(c, s):
        for u in range(CB):
            accs = [jnp.zeros((LANES,), jnp.float32) for _ in range(4)]
            for half in range(NF):
                for j in range(L):
                    accs[j % 4] = accs[j % 4] + rows_v[s, half, u * L + j]
            acc = (accs[0] + accs[1]) + (accs[2] + accs[3])
            out_v[c * CB + u] = acc * scale + bias

    for s in range(NBUF):
        start(s, s)

    def body(r, carry):
        for s in range(NBUF):
            c = r * NBUF + s
            wait(c, s)
            reduce_chunk(c, s)

            @pl.when(c + NBUF < NCHUNK)
            def _():
                start(c + NBUF, s)

        return carry

    lax.fori_loop(0, NCHUNK // NBUF, body, 0)
    pltpu.sync_copy(out_v, out_hbm.at[pl.ds(base, BPW)])


_pool = functools.partial(
    pl.kernel,
    out_type=jax.ShapeDtypeStruct((B, LANES), jnp.float32),
    mesh=plsc.VectorSubcoreMesh(core_axis_name="c", subcore_axis_name="s"),
    scratch_types=[
        pltpu.VMEM((NF, NCHUNK, CB * L), jnp.int32),
        pltpu.VMEM((NBUF, NF, CB * L, LANES), jnp.float32),
        pltpu.VMEM((BPW, LANES), jnp.float32),
        pltpu.VMEM((LANES,), jnp.float32),
        pltpu.SemaphoreType.DMA,
        pltpu.SemaphoreType.DMA,
        pltpu.SemaphoreType.DMA,
        pltpu.SemaphoreType.DMA,
    ],
    compiler_params=pltpu.CompilerParams(use_tc_tiling_on_sc=False),
)(_pool_body)


def kernel(x, embed_table, W_fc, b_fc):
    w0p = jnp.zeros((D, LANES), jnp.float32).at[:, :NCLS].set(W_fc[:D])
    w1p = jnp.zeros((D, LANES), jnp.float32).at[:, :NCLS].set(W_fc[D:])
    eye8 = jnp.eye(PACK, dtype=jnp.float32)
    wbd0 = jnp.kron(eye8, w0p)
    wbd1 = jnp.kron(eye8, w1p)
    bpad = jnp.zeros((LANES,), jnp.float32).at[:NCLS].set(b_fc)
    e512 = embed_table.reshape(_VR, PACK * D)
    t0p, t1p = _project(e512, wbd0, wbd1)
    t0 = t0p.reshape(VOCAB, LANES)
    t1 = t1p.reshape(VOCAB, LANES)
    x3 = x.reshape(NF, NW * NCHUNK, CB * L)
    out16 = _pool(t0, t1, x3, bpad)
    return out16[:, :NCLS]
